# Initial kernel scaffold; baseline (speedup 1.0000x reference)
import jax
import jax.numpy as jnp
from jax.experimental import pallas as pl


def kernel(x, weight):
    raise NotImplementedError("write your pallas kernel here")


# speedup vs baseline: 1.0028x; 1.0028x over previous
import functools

import jax
import jax.numpy as jnp
from jax.experimental import pallas as pl
from jax.experimental.pallas import tpu as pltpu

_LANES = 128
_PACK = 32
_TB = 4096


def _kernel_body(x_ref, w_ref, o_ref, wb_ref):
    @pl.when(pl.program_id(1) == 0)
    def _build_block_diag():
        r = jax.lax.broadcasted_iota(jnp.int32, (_LANES, _LANES), 0)
        c = jax.lax.broadcasted_iota(jnp.int32, (_LANES, _LANES), 1)
        rm = r & 3
        cm = c & 3
        acc = jnp.zeros((_LANES, _LANES), jnp.float32)
        for o in range(4):
            for i in range(4):
                acc = acc + jnp.where((rm == i) & (cm == o), w_ref[o, i], 0.0)
        wb_ref[...] = jnp.where((r >> 2) == (c >> 2), acc, 0.0)

    o_ref[...] = jnp.dot(
        x_ref[...], wb_ref[...], preferred_element_type=jnp.float32
    ).astype(o_ref.dtype)


@functools.partial(jax.jit, static_argnames=())
def kernel(x, weight):
    B, IN = x.shape
    OUT = weight.shape[0]
    Bp = pl.cdiv(B, _PACK) * _PACK
    x_pad = x if Bp == B else jnp.pad(x, ((0, Bp - B), (0, 0)))
    R = Bp // _PACK
    x_packed = x_pad.reshape(R, _LANES)

    tb = min(_TB, R)
    Rg = pl.cdiv(R, 2 * tb) * 2 * tb
    if Rg != R:
        x_packed = jnp.pad(x_packed, ((0, Rg - R), (0, 0)))
    nt = Rg // (2 * tb)

    y_packed = pl.pallas_call(
        _kernel_body,
        out_shape=jax.ShapeDtypeStruct((Rg, _LANES), x.dtype),
        grid=(2, nt),
        in_specs=[
            pl.BlockSpec((tb, _LANES), lambda c, j, _nt=nt: (c * _nt + j, 0)),
            pl.BlockSpec(memory_space=pltpu.MemorySpace.SMEM),
        ],
        out_specs=pl.BlockSpec(
            (tb, _LANES), lambda c, j, _nt=nt: (c * _nt + j, 0)
        ),
        scratch_shapes=[pltpu.VMEM((_LANES, _LANES), jnp.float32)],
        compiler_params=pltpu.CompilerParams(
            dimension_semantics=("parallel", "arbitrary"),
        ),
    )(x_packed, weight)

    return y_packed[:R].reshape(Bp, OUT)[:B]


# speedup vs baseline: 2.6075x; 2.6002x over previous
import functools

import jax
import jax.numpy as jnp
from jax.experimental import pallas as pl
from jax.experimental.pallas import tpu as pltpu

_BT = 8192


def _kernel_body(x_ref, w_ref, o_ref):
    o_ref[...] = jax.lax.dot_general(
        x_ref[...],
        w_ref[...],
        dimension_numbers=(((1,), (1,)), ((), ())),
        preferred_element_type=jnp.float32,
    ).astype(o_ref.dtype)


@functools.partial(jax.jit, static_argnames=())
def kernel(x, weight):
    B, IN = x.shape
    bt = min(_BT, B)
    Bg = pl.cdiv(B, 2 * bt) * 2 * bt
    x_p = x if Bg == B else jnp.pad(x, ((0, Bg - B), (0, 0)))
    nt = Bg // (2 * bt)

    y = pl.pallas_call(
        _kernel_body,
        out_shape=jax.ShapeDtypeStruct((Bg, IN), x.dtype),
        grid=(2, nt),
        in_specs=[
            pl.BlockSpec((bt, IN), lambda c, j, _nt=nt: (c * _nt + j, 0)),
            pl.BlockSpec(memory_space=pltpu.MemorySpace.VMEM),
        ],
        out_specs=pl.BlockSpec((bt, IN), lambda c, j, _nt=nt: (c * _nt + j, 0)),
        compiler_params=pltpu.CompilerParams(
            dimension_semantics=("parallel", "arbitrary"),
        ),
    )(x_p, weight)

    return y[:B]


# speedup vs baseline: 162.5781x; 62.3494x over previous
import functools

import jax
import jax.numpy as jnp
from jax.experimental import pallas as pl
from jax.experimental.pallas import tpu as pltpu

_LANES = 128
_BU = 512


def _build_block_diag(w_ref, a_ref):
    r = jax.lax.broadcasted_iota(jnp.int32, (_LANES, _LANES), 0)
    c = jax.lax.broadcasted_iota(jnp.int32, (_LANES, _LANES), 1)
    rm = r & 3
    cm = c & 3
    acc = jnp.zeros((_LANES, _LANES), jnp.float32)
    for o in range(4):
        for i in range(4):
            acc = acc + jnp.where((rm == o) & (cm == i), w_ref[o, i], 0.0)
    a_ref[...] = jnp.where((r >> 2) == (c >> 2), acc, 0.0)


def _packed_body(x_ref, w_ref, o_ref, a_ref):
    @pl.when(pl.program_id(1) == 0)
    def _init():
        _build_block_diag(w_ref, a_ref)

    s = _BU * 8
    xb = x_ref[...].reshape(s, _LANES)
    a = a_ref[...]
    chunks = [
        jnp.dot(a, xb[c * _LANES:(c + 1) * _LANES, :],
                preferred_element_type=jnp.float32)
        for c in range(s // _LANES)
    ]
    o_ref[...] = jnp.concatenate(chunks, axis=0).reshape(_BU, 8, _LANES)


def _narrow_kernel_body(x_ref, w_ref, o_ref):
    o_ref[...] = jax.lax.dot_general(
        x_ref[...],
        w_ref[...],
        dimension_numbers=(((1,), (1,)), ((), ())),
        preferred_element_type=jnp.float32,
    ).astype(o_ref.dtype)


def _narrow_path(x, weight):
    B, IN = x.shape
    bt = min(8192, B)
    Bg = pl.cdiv(B, 2 * bt) * 2 * bt
    x_p = x if Bg == B else jnp.pad(x, ((0, Bg - B), (0, 0)))
    nt = Bg // (2 * bt)
    y = pl.pallas_call(
        _narrow_kernel_body,
        out_shape=jax.ShapeDtypeStruct((Bg, IN), x.dtype),
        grid=(2, nt),
        in_specs=[
            pl.BlockSpec((bt, IN), lambda c, j, _nt=nt: (c * _nt + j, 0)),
            pl.BlockSpec(memory_space=pltpu.MemorySpace.VMEM),
        ],
        out_specs=pl.BlockSpec((bt, IN), lambda c, j, _nt=nt: (c * _nt + j, 0)),
        compiler_params=pltpu.CompilerParams(
            dimension_semantics=("parallel", "arbitrary"),
        ),
    )(x_p, weight)
    return y[:B]


@functools.partial(jax.jit, static_argnames=())
def kernel(x, weight):
    B, IN = x.shape
    rows_per_tile = 256 * _BU
    if IN != 4 or B % (2 * rows_per_tile) != 0:
        return _narrow_path(x, weight)

    nu = B // 256
    nt = nu // (2 * _BU)
    xv = x.reshape(B // 128, 128, 4).transpose(0, 2, 1).reshape(nu, 8, _LANES)

    yv = pl.pallas_call(
        _packed_body,
        out_shape=jax.ShapeDtypeStruct((nu, 8, _LANES), x.dtype),
        grid=(2, nt),
        in_specs=[
            pl.BlockSpec((_BU, 8, _LANES),
                         lambda c, j, _nt=nt: (c * _nt + j, 0, 0)),
            pl.BlockSpec(memory_space=pltpu.MemorySpace.SMEM),
        ],
        out_specs=pl.BlockSpec((_BU, 8, _LANES),
                               lambda c, j, _nt=nt: (c * _nt + j, 0, 0)),
        scratch_shapes=[pltpu.VMEM((_LANES, _LANES), jnp.float32)],
        compiler_params=pltpu.CompilerParams(
            dimension_semantics=("parallel", "arbitrary"),
        ),
    )(xv, weight)

    return yv.reshape(B // 128, 4, 128).transpose(0, 2, 1).reshape(B, IN)


# speedup vs baseline: 185.2358x; 1.1394x over previous
import functools

import jax
import jax.numpy as jnp
from jax.experimental import pallas as pl
from jax.experimental.pallas import tpu as pltpu

_LANES = 128
_BU = 1024


def _build_block_diag(w_ref, a_ref):
    r = jax.lax.broadcasted_iota(jnp.int32, (_LANES, _LANES), 0)
    c = jax.lax.broadcasted_iota(jnp.int32, (_LANES, _LANES), 1)
    rm = r & 3
    cm = c & 3
    acc = jnp.zeros((_LANES, _LANES), jnp.float32)
    for o in range(4):
        for i in range(4):
            acc = acc + jnp.where((rm == o) & (cm == i), w_ref[o, i], 0.0)
    a_ref[...] = jnp.where((r >> 2) == (c >> 2), acc, 0.0)


def _packed_body(x_ref, w_ref, o_ref, a_ref):
    @pl.when(pl.program_id(1) == 0)
    def _init():
        _build_block_diag(w_ref, a_ref)

    s = _BU * 8
    xb = x_ref[...].reshape(s, _LANES)
    a = a_ref[...]
    chunks = [
        jnp.dot(a, xb[c * _LANES:(c + 1) * _LANES, :],
                preferred_element_type=jnp.float32)
        for c in range(s // _LANES)
    ]
    o_ref[...] = jnp.concatenate(chunks, axis=0).reshape(_BU, 8, _LANES)


def _narrow_kernel_body(x_ref, w_ref, o_ref):
    o_ref[...] = jax.lax.dot_general(
        x_ref[...],
        w_ref[...],
        dimension_numbers=(((1,), (1,)), ((), ())),
        preferred_element_type=jnp.float32,
    ).astype(o_ref.dtype)


def _narrow_path(x, weight):
    B, IN = x.shape
    bt = min(8192, B)
    Bg = pl.cdiv(B, 2 * bt) * 2 * bt
    x_p = x if Bg == B else jnp.pad(x, ((0, Bg - B), (0, 0)))
    nt = Bg // (2 * bt)
    y = pl.pallas_call(
        _narrow_kernel_body,
        out_shape=jax.ShapeDtypeStruct((Bg, IN), x.dtype),
        grid=(2, nt),
        in_specs=[
            pl.BlockSpec((bt, IN), lambda c, j, _nt=nt: (c * _nt + j, 0)),
            pl.BlockSpec(memory_space=pltpu.MemorySpace.VMEM),
        ],
        out_specs=pl.BlockSpec((bt, IN), lambda c, j, _nt=nt: (c * _nt + j, 0)),
        compiler_params=pltpu.CompilerParams(
            dimension_semantics=("parallel", "arbitrary"),
        ),
    )(x_p, weight)
    return y[:B]


@functools.partial(jax.jit, static_argnames=())
def kernel(x, weight):
    B, IN = x.shape
    rows_per_tile = 256 * _BU
    if IN != 4 or B % (2 * rows_per_tile) != 0:
        return _narrow_path(x, weight)

    nu = B // 256
    nt = nu // (2 * _BU)
    xv = x.reshape(B // 128, 128, 4).transpose(0, 2, 1).reshape(nu, 8, _LANES)

    yv = pl.pallas_call(
        _packed_body,
        out_shape=jax.ShapeDtypeStruct((nu, 8, _LANES), x.dtype),
        grid=(2, nt),
        in_specs=[
            pl.BlockSpec((_BU, 8, _LANES),
                         lambda c, j, _nt=nt: (c * _nt + j, 0, 0)),
            pl.BlockSpec(memory_space=pltpu.MemorySpace.SMEM),
        ],
        out_specs=pl.BlockSpec((_BU, 8, _LANES),
                               lambda c, j, _nt=nt: (c * _nt + j, 0, 0)),
        scratch_shapes=[pltpu.VMEM((_LANES, _LANES), jnp.float32)],
        compiler_params=pltpu.CompilerParams(
            dimension_semantics=("parallel", "arbitrary"),
        ),
    )(xv, weight)

    return yv.reshape(B // 128, 4, 128).transpose(0, 2, 1).reshape(B, IN)


# speedup vs baseline: 197.6754x; 1.0672x over previous
import functools

import jax
import jax.numpy as jnp
from jax.experimental import pallas as pl
from jax.experimental.pallas import tpu as pltpu

_LANES = 128
_BU = 2048


def _build_block_diag(w_ref, a_ref):
    r = jax.lax.broadcasted_iota(jnp.int32, (_LANES, _LANES), 0)
    c = jax.lax.broadcasted_iota(jnp.int32, (_LANES, _LANES), 1)
    rm = r & 3
    cm = c & 3
    acc = jnp.zeros((_LANES, _LANES), jnp.float32)
    for o in range(4):
        for i in range(4):
            acc = acc + jnp.where((rm == o) & (cm == i), w_ref[o, i], 0.0)
    a_ref[...] = jnp.where((r >> 2) == (c >> 2), acc, 0.0)


def _packed_body(x_ref, w_ref, o_ref, a_ref):
    @pl.when(pl.program_id(1) == 0)
    def _init():
        _build_block_diag(w_ref, a_ref)

    s = _BU * 8
    xb = x_ref[...].reshape(s, _LANES)
    a = a_ref[...]
    chunks = [
        jnp.dot(a, xb[c * _LANES:(c + 1) * _LANES, :],
                preferred_element_type=jnp.float32)
        for c in range(s // _LANES)
    ]
    o_ref[...] = jnp.concatenate(chunks, axis=0).reshape(_BU, 8, _LANES)


def _narrow_kernel_body(x_ref, w_ref, o_ref):
    o_ref[...] = jax.lax.dot_general(
        x_ref[...],
        w_ref[...],
        dimension_numbers=(((1,), (1,)), ((), ())),
        preferred_element_type=jnp.float32,
    ).astype(o_ref.dtype)


def _narrow_path(x, weight):
    B, IN = x.shape
    bt = min(8192, B)
    Bg = pl.cdiv(B, 2 * bt) * 2 * bt
    x_p = x if Bg == B else jnp.pad(x, ((0, Bg - B), (0, 0)))
    nt = Bg // (2 * bt)
    y = pl.pallas_call(
        _narrow_kernel_body,
        out_shape=jax.ShapeDtypeStruct((Bg, IN), x.dtype),
        grid=(2, nt),
        in_specs=[
            pl.BlockSpec((bt, IN), lambda c, j, _nt=nt: (c * _nt + j, 0)),
            pl.BlockSpec(memory_space=pltpu.MemorySpace.VMEM),
        ],
        out_specs=pl.BlockSpec((bt, IN), lambda c, j, _nt=nt: (c * _nt + j, 0)),
        compiler_params=pltpu.CompilerParams(
            dimension_semantics=("parallel", "arbitrary"),
        ),
    )(x_p, weight)
    return y[:B]


@functools.partial(jax.jit, static_argnames=())
def kernel(x, weight):
    B, IN = x.shape
    rows_per_tile = 256 * _BU
    if IN != 4 or B % (2 * rows_per_tile) != 0:
        return _narrow_path(x, weight)

    nu = B // 256
    nt = nu // (2 * _BU)
    xv = x.reshape(B // 128, 128, 4).transpose(0, 2, 1).reshape(nu, 8, _LANES)

    yv = pl.pallas_call(
        _packed_body,
        out_shape=jax.ShapeDtypeStruct((nu, 8, _LANES), x.dtype),
        grid=(2, nt),
        in_specs=[
            pl.BlockSpec((_BU, 8, _LANES),
                         lambda c, j, _nt=nt: (c * _nt + j, 0, 0)),
            pl.BlockSpec(memory_space=pltpu.MemorySpace.SMEM),
        ],
        out_specs=pl.BlockSpec((_BU, 8, _LANES),
                               lambda c, j, _nt=nt: (c * _nt + j, 0, 0)),
        scratch_shapes=[pltpu.VMEM((_LANES, _LANES), jnp.float32)],
        compiler_params=pltpu.CompilerParams(
            dimension_semantics=("parallel", "arbitrary"),
        ),
    )(xv, weight)

    return yv.reshape(B // 128, 4, 128).transpose(0, 2, 1).reshape(B, IN)
